# baseline (speedup 1.0000x reference)
import functools

import jax
import jax.numpy as jnp
from jax.experimental import pallas as pl

_K = 5
_KK = _K * _K


def _natt_kernel(xp_ref, xc_ref, xn_ref, wq_ref, wk_ref, b_ref, rpb_ref,
                 out_ref, *, BH, H, W, DIM, HEADS, HD):
    pid = pl.program_id(0)

    xc = xc_ref[0].reshape(BH * W, DIM)
    q = jnp.dot(xc, wq_ref[...], preferred_element_type=jnp.float32)
    q = (q + b_ref[:, :DIM]).reshape(BH, W, DIM)

    def kproj(ref, rows):
        xv = ref[0].reshape(rows * W, DIM)
        kv = jnp.dot(xv, wk_ref[...], preferred_element_type=jnp.float32)
        return (kv + b_ref[:, DIM:]).reshape(rows, W, DIM)

    kstrip = jnp.concatenate(
        [kproj(xp_ref, 2), kproj(xc_ref, BH), kproj(xn_ref, 2)], axis=0)

    iog = jax.lax.broadcasted_iota(jnp.int32, (BH, W), 0) + pid * BH
    jog = jax.lax.broadcasted_iota(jnp.int32, (BH, W), 1)
    i3 = iog[..., None]
    j3 = jog[..., None]

    for h in range(HEADS):
        qh = q[:, :, h * HD:(h + 1) * HD]
        kh = kstrip[:, :, h * HD:(h + 1) * HD]
        for ki in range(_K):
            base = kh[ki:BH + ki]
            top = kh[ki + 2][None]
            bot = kh[BH - 3 + ki][None]
            kr = jnp.where(i3 < 2, top,
                           jnp.where(i3 > H - 3, bot, base))
            for kj in range(_K):
                sh = kr if kj == 2 else jnp.roll(kr, 2 - kj, axis=1)
                left = kr[:, kj][:, None, :]
                right = kr[:, W - 5 + kj][:, None, :]
                kk = jnp.where(j3 < 2, left,
                               jnp.where(j3 > W - 3, right, sh))
                logit = jnp.sum(qh * kk, axis=-1)

                def _colbias(a, _kj=kj, _h=h):
                    cint = rpb_ref[_h, a, 2 + _kj]
                    c0 = rpb_ref[_h, a, 4 + _kj]
                    c1 = rpb_ref[_h, a, 3 + _kj]
                    c2 = rpb_ref[_h, a, 1 + _kj]
                    c3 = rpb_ref[_h, a, _kj]
                    return jnp.where(jog == 0, c0,
                           jnp.where(jog == 1, c1,
                           jnp.where(jog == W - 2, c2,
                           jnp.where(jog == W - 1, c3, cint))))

                bias = jnp.where(iog == 0, _colbias(4 + ki),
                       jnp.where(iog == 1, _colbias(3 + ki),
                       jnp.where(iog == H - 2, _colbias(1 + ki),
                       jnp.where(iog == H - 1, _colbias(ki),
                                 _colbias(2 + ki)))))
                out_ref[0, h, :, :, ki * _K + kj] = logit + bias


def kernel(x, qk_w, qk_b, rpb):
    B, H, W, DIM = x.shape
    HEADS = rpb.shape[0]
    HD = DIM // HEADS
    BH = 8
    NB = H // BH

    wt = qk_w.T
    wq = wt[:, :DIM]
    wk = wt[:, DIM:]
    b2 = qk_b.reshape(1, 2 * DIM)

    body = functools.partial(_natt_kernel, BH=BH, H=H, W=W, DIM=DIM,
                             HEADS=HEADS, HD=HD)

    out = pl.pallas_call(
        body,
        grid=(NB,),
        in_specs=[
            pl.BlockSpec((1, 2, W, DIM),
                         lambda i: (0, jnp.maximum((BH // 2) * i - 1, 0),
                                    0, 0)),
            pl.BlockSpec((1, BH, W, DIM), lambda i: (0, i, 0, 0)),
            pl.BlockSpec((1, 2, W, DIM),
                         lambda i: (0, jnp.minimum((BH // 2) * (i + 1),
                                                   H // 2 - 1), 0, 0)),
            pl.BlockSpec((DIM, DIM), lambda i: (0, 0)),
            pl.BlockSpec((DIM, DIM), lambda i: (0, 0)),
            pl.BlockSpec((1, 2 * DIM), lambda i: (0, 0)),
            pl.BlockSpec(rpb.shape, lambda i: (0, 0, 0)),
        ],
        out_specs=pl.BlockSpec((1, HEADS, BH, W, _KK),
                               lambda i: (0, 0, i, 0, 0)),
        out_shape=jax.ShapeDtypeStruct((B, HEADS, H, W, _KK), jnp.float32),
    )(x, x, x, wq, wk, b2, rpb)
    return out


# speedup vs baseline: 7.6575x; 7.6575x over previous
import functools

import jax
import jax.numpy as jnp
from jax.experimental import pallas as pl

_K = 5
_KK = _K * _K


def _natt_kernel(xp_ref, xc_ref, xn_ref, wq_ref, wk_ref, bq_ref, bk_ref,
                 rpb_ref, out_ref, *, BH, H, W, DIM, HEADS, HD):
    pid = pl.program_id(0)

    xc = xc_ref[0].reshape(DIM, BH * W)
    qT = jnp.dot(wq_ref[...], xc, preferred_element_type=jnp.float32)
    qT = (qT + bq_ref[...]).reshape(DIM, BH, W)

    x3 = jnp.concatenate(
        [xp_ref[0].reshape(DIM, BH * W), xc, xn_ref[0].reshape(DIM, BH * W)],
        axis=1)
    kT = jnp.dot(wk_ref[...], x3, preferred_element_type=jnp.float32)
    kT = (kT + bk_ref[...]).reshape(DIM, 3 * BH, W)

    iog = jax.lax.broadcasted_iota(jnp.int32, (BH, W), 0) + pid * BH
    jog = jax.lax.broadcasted_iota(jnp.int32, (BH, W), 1)
    i3 = iog[None]
    j3 = jog[None]

    for h in range(HEADS):
        qh = qT[h * HD:(h + 1) * HD]
        kh = kT[h * HD:(h + 1) * HD]
        for ki in range(_K):
            base = kh[:, BH + ki - 2:2 * BH + ki - 2]
            top = kh[:, BH + ki][:, None]
            bot = kh[:, 2 * BH - 5 + ki][:, None]
            kr = jnp.where(i3 < 2, top,
                           jnp.where(i3 > H - 3, bot, base))
            for kj in range(_K):
                sh = kr if kj == 2 else jnp.roll(kr, 2 - kj, axis=2)
                left = kr[:, :, kj][:, :, None]
                right = kr[:, :, W - 5 + kj][:, :, None]
                kk = jnp.where(j3 < 2, left,
                               jnp.where(j3 > W - 3, right, sh))
                logit = jnp.sum(qh * kk, axis=0)

                def _colbias(a, _kj=kj, _h=h):
                    cint = rpb_ref[_h, a, 2 + _kj]
                    c0 = rpb_ref[_h, a, 4 + _kj]
                    c1 = rpb_ref[_h, a, 3 + _kj]
                    c2 = rpb_ref[_h, a, 1 + _kj]
                    c3 = rpb_ref[_h, a, _kj]
                    return jnp.where(jog == 0, c0,
                           jnp.where(jog == 1, c1,
                           jnp.where(jog == W - 2, c2,
                           jnp.where(jog == W - 1, c3, cint))))

                bias = jnp.where(iog == 0, _colbias(4 + ki),
                       jnp.where(iog == 1, _colbias(3 + ki),
                       jnp.where(iog == H - 2, _colbias(1 + ki),
                       jnp.where(iog == H - 1, _colbias(ki),
                                 _colbias(2 + ki)))))
                out_ref[0, h, ki * _K + kj] = logit + bias


def kernel(x, qk_w, qk_b, rpb):
    B, H, W, DIM = x.shape
    HEADS = rpb.shape[0]
    HD = DIM // HEADS
    BH = 8
    NB = H // BH

    xT = x.transpose(0, 3, 1, 2)
    wq = qk_w[:DIM]
    wk = qk_w[DIM:]
    bq = qk_b[:DIM].reshape(DIM, 1)
    bk = qk_b[DIM:].reshape(DIM, 1)

    body = functools.partial(_natt_kernel, BH=BH, H=H, W=W, DIM=DIM,
                             HEADS=HEADS, HD=HD)

    out = pl.pallas_call(
        body,
        grid=(NB,),
        in_specs=[
            pl.BlockSpec((1, DIM, BH, W),
                         lambda i: (0, 0, jnp.maximum(i - 1, 0), 0)),
            pl.BlockSpec((1, DIM, BH, W), lambda i: (0, 0, i, 0)),
            pl.BlockSpec((1, DIM, BH, W),
                         lambda i: (0, 0, jnp.minimum(i + 1, NB - 1), 0)),
            pl.BlockSpec((DIM, DIM), lambda i: (0, 0)),
            pl.BlockSpec((DIM, DIM), lambda i: (0, 0)),
            pl.BlockSpec((DIM, 1), lambda i: (0, 0)),
            pl.BlockSpec((DIM, 1), lambda i: (0, 0)),
            pl.BlockSpec(rpb.shape, lambda i: (0, 0, 0)),
        ],
        out_specs=pl.BlockSpec((1, HEADS, _KK, BH, W),
                               lambda i: (0, 0, 0, i, 0)),
        out_shape=jax.ShapeDtypeStruct((B, HEADS, _KK, H, W), jnp.float32),
    )(xT, xT, xT, wq, wk, bq, bk, rpb)
    return out.transpose(0, 1, 3, 4, 2)
